# baseline (speedup 1.0000x reference)
import functools

import jax
import jax.numpy as jnp
from jax import lax
from jax.experimental import pallas as pl
from jax.experimental.pallas import tpu as pltpu
from jax.experimental.pallas import tpu_sc as plsc

NUM_SELECT = 32
NUM_CORES = 2
NUM_SUBCORES = 16
LANES = 16
NEG = -3.0e38


def _scores_body(l_ref, s_ref):
    lt = l_ref[0]
    K = lt.shape[0]
    kiota = lax.broadcasted_iota(jnp.int32, (K, 1), 0)
    colsum = jnp.sum(lt, axis=1, keepdims=True)
    cmax = jnp.max(colsum)
    m = jnp.min(jnp.where(colsum == cmax, kiota, K))
    rowmax = jnp.max(lt, axis=0, keepdims=True)
    e = jnp.exp(lt - rowmax)
    denom = jnp.sum(e, axis=0, keepdims=True)
    num = jnp.sum(jnp.where(kiota == m, e, 0.0), axis=0, keepdims=True)
    s_ref[0, 0] = (num / denom)[0]


def _scores(logits):
    B, S, K = logits.shape
    lt = jnp.transpose(logits, (0, 2, 1))
    out = pl.pallas_call(
        _scores_body,
        grid=(B,),
        in_specs=[pl.BlockSpec((1, K, S), lambda b: (b, 0, 0))],
        out_specs=pl.BlockSpec((1, 1, S), lambda b: (b, 0, 0)),
        out_shape=jax.ShapeDtypeStruct((B, 1, S), jnp.float32),
    )(lt)
    return out.reshape(B * S)


def _shuffle(v, perm):
    dnums = lax.GatherDimensionNumbers(
        offset_dims=(), collapsed_slice_dims=(0,), start_index_map=(0,))
    return lax.gather(v, perm[:, None], dnums, (1,),
                      mode=lax.GatherScatterMode.PROMISE_IN_BOUNDS)


def _allmax(v, lane):
    for d in (1, 2, 4, 8):
        v = jnp.maximum(v, _shuffle(v, lane ^ d))
    return v


def _allmin_i32(v, lane):
    for d in (1, 2, 4, 8):
        v = jnp.minimum(v, _shuffle(v, lane ^ d))
    return v


def _stable_top32(score_ref, nvregs, emit):
    lane = lax.broadcasted_iota(jnp.int32, (LANES,), 0)
    mask0 = lane == 0
    nacc = min(8, nvregs)

    def rank_body(rank, _):
        bv = [jnp.full((LANES,), NEG, jnp.float32) for _ in range(nacc)]
        bi = [jnp.zeros((LANES,), jnp.int32) for _ in range(nacc)]
        for i in range(nvregs):
            j = i % nacc
            v = score_ref[pl.ds(i * LANES, LANES)]
            upd = v > bv[j]
            bv[j] = jnp.where(upd, v, bv[j])
            bi[j] = jnp.where(upd, i, bi[j])

        def comb(a, b):
            av, ai = a
            bv_, bi_ = b
            upd = (bv_ > av) | ((bv_ == av) & (bi_ < ai))
            return jnp.where(upd, bv_, av), jnp.where(upd, bi_, ai)

        accs = list(zip(bv, bi))
        while len(accs) > 1:
            accs = [comb(accs[k], accs[k + 1])
                    for k in range(0, len(accs) - 1, 2)] + (
                       [accs[-1]] if len(accs) % 2 else [])
        fv, fi = accs[0]
        vmax = _allmax(fv, lane)
        combo = fi * LANES + lane
        idx = _allmin_i32(jnp.where(fv == vmax, combo, nvregs * LANES), lane)
        emit(rank, idx, vmax)
        plsc.store_scatter(score_ref, [idx],
                           jnp.full((LANES,), NEG, jnp.float32), mask=mask0)
        return 0

    lax.fori_loop(0, NUM_SELECT, rank_body, 0)


def _make_selector(B, S, C):
    n_tiles = NUM_CORES * NUM_SUBCORES
    chunk = (B * S) // n_tiles
    nv = chunk // LANES
    b_per_core = B // NUM_CORES
    chunks_per_b = NUM_SUBCORES // b_per_core
    npool = chunks_per_b * NUM_SELECT
    mesh = plsc.VectorSubcoreMesh(core_axis_name="c", subcore_axis_name="s")

    @functools.partial(
        pl.kernel, mesh=mesh,
        compiler_params=pltpu.CompilerParams(needs_layout_passes=False),
        out_type=jax.ShapeDtypeStruct((B * NUM_SELECT, C), jnp.float32),
        scratch_types=[
            pltpu.VMEM((chunk,), jnp.float32),
            pltpu.VMEM((NUM_SELECT,), jnp.float32),
            pltpu.VMEM((NUM_SELECT,), jnp.int32),
            pltpu.VMEM((npool,), jnp.float32),
            pltpu.VMEM((npool,), jnp.int32),
            pltpu.VMEM((NUM_SELECT,), jnp.int32),
            pltpu.VMEM((NUM_SELECT, C), jnp.float32),
            pltpu.VMEM_SHARED((b_per_core, npool), jnp.float32),
            pltpu.VMEM_SHARED((b_per_core, npool), jnp.int32),
            pltpu.SemaphoreType.DMA,
        ],
    )
    def selector(scores_hbm, x_hbm, out_hbm,
                 sv, cv, ci, mv, mi, si, rows, shv, shi, sem):
        core = lax.axis_index("c")
        sid = lax.axis_index("s")
        bl = sid // chunks_per_b
        b = core * b_per_core + bl
        ch = sid % chunks_per_b
        base = b * S + ch * chunk
        lane = lax.broadcasted_iota(jnp.int32, (LANES,), 0)
        mask0 = lane == 0

        pltpu.sync_copy(scores_hbm.at[pl.ds(base, chunk)], sv)

        def emit_local(rank, idx, vmax):
            r = jnp.full((LANES,), rank, jnp.int32)
            plsc.store_scatter(cv, [r], vmax, mask=mask0)
            plsc.store_scatter(ci, [r], base + idx, mask=mask0)

        _stable_top32(sv, nv, emit_local)

        pltpu.sync_copy(cv, shv.at[bl, pl.ds(ch * NUM_SELECT, NUM_SELECT)])
        pltpu.sync_copy(ci, shi.at[bl, pl.ds(ch * NUM_SELECT, NUM_SELECT)])
        plsc.subcore_barrier()

        @pl.when(ch == 0)
        def _merge():
            pltpu.sync_copy(shv.at[bl], mv)
            pltpu.sync_copy(shi.at[bl], mi)

            def emit_final(rank, p, vmax):
                row = plsc.load_gather(mi, [p])
                plsc.store_scatter(si, [jnp.full((LANES,), rank, jnp.int32)],
                                   row, mask=mask0)

            _stable_top32(mv, npool // LANES, emit_final)
            pltpu.async_copy(x_hbm.at[si], rows, sem).wait()
            pltpu.sync_copy(rows, out_hbm.at[pl.ds(b * NUM_SELECT, NUM_SELECT)])

    return selector


def kernel(x, logits):
    B, S, C = x.shape
    scores = _scores(logits)
    selector = _make_selector(B, S, C)
    out = selector(scores, x.reshape(B * S, C))
    return out.reshape(B, NUM_SELECT, C)


# speedup vs baseline: 1.0377x; 1.0377x over previous
import functools

import jax
import jax.numpy as jnp
from jax import lax
from jax.experimental import pallas as pl
from jax.experimental.pallas import tpu as pltpu
from jax.experimental.pallas import tpu_sc as plsc

NUM_SELECT = 32
NUM_CORES = 2
NUM_SUBCORES = 16
LANES = 16
NEG = -3.0e38


def _scores_body(l_ref, s_ref):
    lt = l_ref[0]
    K = lt.shape[0]
    kiota = lax.broadcasted_iota(jnp.int32, (K, 1), 0)
    colsum = jnp.sum(lt, axis=1, keepdims=True)
    cmax = jnp.max(colsum)
    m = jnp.min(jnp.where(colsum == cmax, kiota, K))
    rowmax = jnp.max(lt, axis=0, keepdims=True)
    e = jnp.exp(lt - rowmax)
    denom = jnp.sum(e, axis=0, keepdims=True)
    ltm = l_ref[0, pl.ds(m, 1), :]
    num = jnp.exp(ltm - rowmax)
    s_ref[0, 0] = (num / denom)[0]


def _scores(logits):
    B, S, K = logits.shape
    lt = jnp.transpose(logits, (0, 2, 1))
    out = pl.pallas_call(
        _scores_body,
        grid=(B,),
        in_specs=[pl.BlockSpec((1, K, S), lambda b: (b, 0, 0))],
        out_specs=pl.BlockSpec((1, 1, S), lambda b: (b, 0, 0)),
        out_shape=jax.ShapeDtypeStruct((B, 1, S), jnp.float32),
    )(lt)
    return out.reshape(B * S)


def _shuffle(v, perm):
    dnums = lax.GatherDimensionNumbers(
        offset_dims=(), collapsed_slice_dims=(0,), start_index_map=(0,))
    return lax.gather(v, perm[:, None], dnums, (1,),
                      mode=lax.GatherScatterMode.PROMISE_IN_BOUNDS)


def _allmax(v, lane):
    for d in (1, 2, 4, 8):
        v = jnp.maximum(v, _shuffle(v, lane ^ d))
    return v


def _allmin_i32(v, lane):
    for d in (1, 2, 4, 8):
        v = jnp.minimum(v, _shuffle(v, lane ^ d))
    return v


def _stable_top32(score_ref, nvregs, emit):
    lane = lax.broadcasted_iota(jnp.int32, (LANES,), 0)
    mask0 = lane == 0
    nacc = min(4, nvregs)

    def comb(a, b):
        av, ai = a
        bv_, bi_ = b
        upd = (bv_ > av) | ((bv_ == av) & (bi_ < ai))
        return jnp.where(upd, bv_, av), jnp.where(upd, bi_, ai)

    bv = [jnp.full((LANES,), NEG, jnp.float32) for _ in range(nacc)]
    bi = [jnp.zeros((LANES,), jnp.int32) for _ in range(nacc)]
    for i in range(nvregs):
        j = i % nacc
        v = score_ref[pl.ds(i * LANES, LANES)]
        upd = v > bv[j]
        bv[j] = jnp.where(upd, v, bv[j])
        bi[j] = jnp.where(upd, i, bi[j])
    accs = list(zip(bv, bi))
    while len(accs) > 1:
        accs = [comb(accs[k], accs[k + 1])
                for k in range(0, len(accs) - 1, 2)] + (
                   [accs[-1]] if len(accs) % 2 else [])

    def rank_body(rank, state):
        fv, fi = state
        vmax = _allmax(fv, lane)
        combo = fi * LANES + lane
        idx = _allmin_i32(jnp.where(fv == vmax, combo, nvregs * LANES), lane)
        emit(rank, idx, vmax)
        plsc.store_scatter(score_ref, [idx],
                           jnp.full((LANES,), NEG, jnp.float32), mask=mask0)
        lsel = idx & (LANES - 1)
        cv, ci = None, None
        for j in range(nvregs // LANES):
            g = plsc.load_gather(score_ref, [(lane + j * LANES) * LANES + lsel])
            gi = lane + j * LANES
            cv, ci = (g, gi) if cv is None else comb((cv, ci), (g, gi))
        nmax = _allmax(cv, lane)
        nidx = _allmin_i32(jnp.where(cv == nmax, ci, nvregs), lane)
        hit = lane == lsel
        return jnp.where(hit, nmax, fv), jnp.where(hit, nidx, fi)

    lax.fori_loop(0, NUM_SELECT, rank_body, accs[0])


def _make_selector(B, S, C):
    n_tiles = NUM_CORES * NUM_SUBCORES
    chunk = (B * S) // n_tiles
    nv = chunk // LANES
    b_per_core = B // NUM_CORES
    chunks_per_b = NUM_SUBCORES // b_per_core
    npool = chunks_per_b * NUM_SELECT
    mesh = plsc.VectorSubcoreMesh(core_axis_name="c", subcore_axis_name="s")

    @functools.partial(
        pl.kernel, mesh=mesh,
        compiler_params=pltpu.CompilerParams(needs_layout_passes=False),
        out_type=jax.ShapeDtypeStruct((B * NUM_SELECT, C), jnp.float32),
        scratch_types=[
            pltpu.VMEM((chunk,), jnp.float32),
            pltpu.VMEM((NUM_SELECT,), jnp.float32),
            pltpu.VMEM((NUM_SELECT,), jnp.int32),
            pltpu.VMEM((npool,), jnp.float32),
            pltpu.VMEM((npool,), jnp.int32),
            pltpu.VMEM((NUM_SELECT,), jnp.int32),
            pltpu.VMEM((NUM_SELECT, C), jnp.float32),
            pltpu.VMEM_SHARED((b_per_core, npool), jnp.float32),
            pltpu.VMEM_SHARED((b_per_core, npool), jnp.int32),
            pltpu.SemaphoreType.DMA,
        ],
    )
    def selector(scores_hbm, x_hbm, out_hbm,
                 sv, cv, ci, mv, mi, si, rows, shv, shi, sem):
        core = lax.axis_index("c")
        sid = lax.axis_index("s")
        bl = sid // chunks_per_b
        b = core * b_per_core + bl
        ch = sid % chunks_per_b
        base = b * S + ch * chunk
        lane = lax.broadcasted_iota(jnp.int32, (LANES,), 0)
        mask0 = lane == 0

        pltpu.sync_copy(scores_hbm.at[pl.ds(base, chunk)], sv)

        def emit_local(rank, idx, vmax):
            r = jnp.full((LANES,), rank, jnp.int32)
            plsc.store_scatter(cv, [r], vmax, mask=mask0)
            plsc.store_scatter(ci, [r], base + idx, mask=mask0)

        _stable_top32(sv, nv, emit_local)

        pltpu.sync_copy(cv, shv.at[bl, pl.ds(ch * NUM_SELECT, NUM_SELECT)])
        pltpu.sync_copy(ci, shi.at[bl, pl.ds(ch * NUM_SELECT, NUM_SELECT)])
        plsc.subcore_barrier()

        @pl.when(ch == 0)
        def _merge():
            pltpu.sync_copy(shv.at[bl], mv)
            pltpu.sync_copy(shi.at[bl], mi)

            def emit_final(rank, p, vmax):
                row = plsc.load_gather(mi, [p])
                plsc.store_scatter(si, [jnp.full((LANES,), rank, jnp.int32)],
                                   row, mask=mask0)

            _stable_top32(mv, npool // LANES, emit_final)
            pltpu.async_copy(x_hbm.at[si], rows, sem).wait()
            pltpu.sync_copy(rows, out_hbm.at[pl.ds(b * NUM_SELECT, NUM_SELECT)])

    return selector


def kernel(x, logits):
    B, S, C = x.shape
    scores = _scores(logits)
    selector = _make_selector(B, S, C)
    out = selector(scores, x.reshape(B * S, C))
    return out.reshape(B, NUM_SELECT, C)
